# baseline (speedup 1.0000x reference)
import functools

import jax
import jax.numpy as jnp
from jax import lax
from jax.experimental import pallas as pl
from jax.experimental.pallas import tpu as pltpu
from jax.experimental.pallas import tpu_sc as plsc


def kernel(batch_hidden_states, lengths):
    B, T, H = batch_hidden_states.shape
    flat = batch_hidden_states.reshape(B * T, H)
    mesh = plsc.VectorSubcoreMesh(
        core_axis_name="c", subcore_axis_name="s", num_cores=1
    )

    @functools.partial(
        pl.kernel,
        mesh=mesh,
        out_type=jax.ShapeDtypeStruct((B, H), jnp.float32),
        scratch_types=[
            pltpu.VMEM((B,), jnp.int32),
            pltpu.VMEM((1, H), jnp.float32),
            pltpu.SemaphoreType.DMA,
        ],
    )
    def gather_last(x_hbm, len_hbm, out_hbm, idx_v, row_v, sem):
        b = lax.axis_index("s")
        pltpu.sync_copy(len_hbm, idx_v)
        lane = lax.iota(jnp.int32, B)
        fidx = jnp.clip(idx_v[...] - 1, 0, T - 1) + lane * T
        idx_v[...] = fidx.at[(lane + b) % B].get(mode="promise_in_bounds")
        pltpu.async_copy(x_hbm.at[idx_v.at[pl.ds(0, 1)]], row_v, sem).wait()
        pltpu.sync_copy(row_v, out_hbm.at[pl.ds(b, 1)])

    return gather_last(flat, lengths.astype(jnp.int32))


# speedup vs baseline: 1.8292x; 1.8292x over previous
import jax
import jax.numpy as jnp
from jax.experimental import pallas as pl
from jax.experimental.pallas import tpu as pltpu


def kernel(batch_hidden_states, lengths):
    B, T, H = batch_hidden_states.shape

    def body(len_ref, x_ref, o_ref):
        b = pl.program_id(0)
        idx = jnp.clip(len_ref[b] - 1, 0, T - 1)
        o_ref[0] = x_ref[0, pl.ds(idx % 8, 1), :]

    def x_map(b, len_ref):
        return (b, jnp.clip(len_ref[b] - 1, 0, T - 1) // 8, 0)

    return pl.pallas_call(
        body,
        grid_spec=pltpu.PrefetchScalarGridSpec(
            num_scalar_prefetch=1,
            grid=(B,),
            in_specs=[pl.BlockSpec((1, 8, H), x_map)],
            out_specs=pl.BlockSpec((1, 1, H), lambda b, len_ref: (b, 0, 0)),
        ),
        out_shape=jax.ShapeDtypeStruct((B, 1, H), jnp.float32),
    )(lengths.astype(jnp.int32), batch_hidden_states)[:, 0, :]


# speedup vs baseline: 8.4299x; 4.6084x over previous
import jax
import jax.numpy as jnp
from jax.experimental import pallas as pl
from jax.experimental.pallas import tpu as pltpu


def kernel(batch_hidden_states, lengths):
    B, T, H = batch_hidden_states.shape

    def body(len_ref, x_hbm, o_ref, sem):
        copies = []
        for b in range(B):
            idx = jnp.clip(len_ref[b] - 1, 0, T - 1)
            copies.append(
                pltpu.make_async_copy(
                    x_hbm.at[b, pl.ds(idx, 1), :],
                    o_ref.at[pl.ds(b, 1), :],
                    sem,
                )
            )
        for c in copies:
            c.start()
        for c in copies:
            c.wait()

    return pl.pallas_call(
        body,
        grid_spec=pltpu.PrefetchScalarGridSpec(
            num_scalar_prefetch=1,
            grid=(1,),
            in_specs=[pl.BlockSpec(memory_space=pl.ANY)],
            out_specs=pl.BlockSpec((B, H), lambda i, len_ref: (0, 0)),
            scratch_shapes=[pltpu.SemaphoreType.DMA],
        ),
        out_shape=jax.ShapeDtypeStruct((B, H), jnp.float32),
    )(lengths.astype(jnp.int32), batch_hidden_states)
